# Initial kernel scaffold; baseline (speedup 1.0000x reference)
import jax
import jax.numpy as jnp
from jax.experimental import pallas as pl


def kernel(edge_index_mp, emb_weight):
    raise NotImplementedError("write your pallas kernel here")


# speedup vs baseline: 11.2861x; 11.2861x over previous
import functools

import jax
import jax.numpy as jnp
from jax import lax
from jax.experimental import pallas as pl
from jax.experimental.pallas import tpu as pltpu
from jax.experimental.pallas import tpu_sc as plsc

N_NODES = 10000
N_FEATS = 128
N_EDGES = 320000
N_LAYERS = 3

NC = 2
NS = 16
NW = NC * NS
CHUNK = 128
EPT = N_EDGES // NW
PAD_ROWS = 240
N_PAD = N_NODES + PAD_ROWS
CPT = (EPT + PAD_ROWS) // CHUNK
RPS = N_PAD // NS
DEGW = 128


def _vector_mesh():
    return plsc.VectorSubcoreMesh(core_axis_name="c", subcore_axis_name="s")


def _sc_degree(dst_t, zdeg, ones):
    @functools.partial(
        pl.kernel,
        out_type=jax.ShapeDtypeStruct((NC, N_PAD, DEGW), jnp.float32),
        mesh=_vector_mesh(),
        scratch_types=[
            pltpu.VMEM((CPT, CHUNK), jnp.int32),
            pltpu.VMEM((CHUNK, DEGW), jnp.float32),
            pltpu.VMEM_SHARED((N_PAD, DEGW), jnp.float32),
        ],
    )
    def k(dst_hbm, z_hbm, o_hbm, out_hbm, dstv, onesv, acc):
        cid = lax.axis_index("c")
        sid = lax.axis_index("s")
        wid = sid * NC + cid
        base = sid * RPS
        pltpu.sync_copy(z_hbm.at[pl.ds(base, RPS)], acc.at[pl.ds(base, RPS)])
        pltpu.sync_copy(o_hbm, onesv)
        pltpu.sync_copy(dst_hbm.at[wid], dstv)
        plsc.subcore_barrier()

        @pl.loop(0, CPT)
        def _(c):
            pltpu.sync_copy(onesv, acc.at[dstv.at[c]], add=True)

        plsc.subcore_barrier()
        pltpu.sync_copy(acc.at[pl.ds(base, RPS)],
                        out_hbm.at[cid, pl.ds(base, RPS)])

    return k(dst_t, zdeg, ones)


def _sc_scatter_rows(y, src_t, dst_t, zrows):
    @functools.partial(
        pl.kernel,
        out_type=jax.ShapeDtypeStruct((NC, N_PAD, N_FEATS), jnp.float32),
        mesh=_vector_mesh(),
        scratch_types=[
            pltpu.VMEM((CPT, CHUNK), jnp.int32),
            pltpu.VMEM((CPT, CHUNK), jnp.int32),
            pltpu.VMEM((CHUNK, N_FEATS), jnp.float32),
            pltpu.VMEM_SHARED((N_PAD, N_FEATS), jnp.float32),
        ],
    )
    def k(y_hbm, src_hbm, dst_hbm, z_hbm, out_hbm, srcv, dstv, rows, acc):
        cid = lax.axis_index("c")
        sid = lax.axis_index("s")
        wid = sid * NC + cid
        base = sid * RPS
        pltpu.sync_copy(z_hbm.at[pl.ds(base, RPS)], acc.at[pl.ds(base, RPS)])
        pltpu.sync_copy(src_hbm.at[wid], srcv)
        pltpu.sync_copy(dst_hbm.at[wid], dstv)
        plsc.subcore_barrier()

        @pl.loop(0, CPT)
        def _(c):
            pltpu.sync_copy(y_hbm.at[srcv.at[c]], rows)
            pltpu.sync_copy(rows, acc.at[dstv.at[c]], add=True)

        plsc.subcore_barrier()
        pltpu.sync_copy(acc.at[pl.ds(base, RPS)],
                        out_hbm.at[cid, pl.ds(base, RPS)])

    return k(y, src_t, dst_t, zrows)


_BR = 256
_GRID = (N_PAD // _BR,)


def _row_spec(w):
    return pl.BlockSpec((_BR, w), lambda i: (i, 0))


def _f32_out():
    return jax.ShapeDtypeStruct((N_PAD, N_FEATS), jnp.float32)


def _tc_prep(dega, degb, x0):
    def body(da, db, x, dis_o, y_o):
        deg = da[:, 0:1] + db[:, 0:1]
        dis = jnp.where(deg > 0.0, lax.rsqrt(jnp.maximum(deg, 1.0)), 0.0)
        disb = jnp.broadcast_to(dis, (_BR, N_FEATS))
        dis_o[...] = disb
        y_o[...] = x[...] * disb

    return pl.pallas_call(
        body,
        grid=_GRID,
        in_specs=[_row_spec(DEGW), _row_spec(DEGW), _row_spec(N_FEATS)],
        out_specs=[_row_spec(N_FEATS), _row_spec(N_FEATS)],
        out_shape=[_f32_out(), _f32_out()],
    )(dega, degb, x0)


def _tc_step(dis, sa, sb, accp):
    def body(d_ref, sa_ref, sb_ref, ap_ref, y_o, acc_o):
        d = d_ref[...]
        ds_ = d * (sa_ref[...] + sb_ref[...])
        y_o[...] = d * ds_
        acc_o[...] = ap_ref[...] + ds_

    return pl.pallas_call(
        body,
        grid=_GRID,
        in_specs=[_row_spec(N_FEATS)] * 4,
        out_specs=[_row_spec(N_FEATS), _row_spec(N_FEATS)],
        out_shape=[_f32_out(), _f32_out()],
    )(dis, sa, sb, accp)


def _tc_last(dis, sa, sb, accp, x0):
    def body(d_ref, sa_ref, sb_ref, ap_ref, x_ref, o_ref):
        ds_ = d_ref[...] * (sa_ref[...] + sb_ref[...])
        o_ref[...] = 0.25 * (x_ref[...] + ap_ref[...] + ds_)

    return pl.pallas_call(
        body,
        grid=_GRID,
        in_specs=[_row_spec(N_FEATS)] * 5,
        out_specs=_row_spec(N_FEATS),
        out_shape=_f32_out(),
    )(dis, sa, sb, accp, x0)


def kernel(edge_index_mp, emb_weight):
    pad = jnp.arange(N_NODES, N_NODES + PAD_ROWS, dtype=jnp.int32)
    padt = jnp.broadcast_to(pad, (NW, PAD_ROWS))
    src_t = jnp.concatenate(
        [edge_index_mp[0].reshape(NW, EPT), padt], axis=1
    ).reshape(NW, CPT, CHUNK)
    dst_t = jnp.concatenate(
        [edge_index_mp[1].reshape(NW, EPT), padt], axis=1
    ).reshape(NW, CPT, CHUNK)
    x0 = jnp.pad(emb_weight, ((0, PAD_ROWS), (0, 0)))
    zrows = jnp.zeros((N_PAD, N_FEATS), jnp.float32)
    zdeg = jnp.zeros((N_PAD, DEGW), jnp.float32)
    ones = jnp.ones((CHUNK, DEGW), jnp.float32)

    degp = _sc_degree(dst_t, zdeg, ones)
    dis, y = _tc_prep(degp[0], degp[1], x0)
    acc = zrows
    out = None
    for layer in range(N_LAYERS):
        s = _sc_scatter_rows(y, src_t, dst_t, zrows)
        if layer < N_LAYERS - 1:
            y, acc = _tc_step(dis, s[0], s[1], acc)
        else:
            out = _tc_last(dis, s[0], s[1], acc, x0)
    return out[:N_NODES]


# speedup vs baseline: 14.3131x; 1.2682x over previous
import functools

import jax
import jax.numpy as jnp
from jax import lax
from jax.experimental import pallas as pl
from jax.experimental.pallas import tpu as pltpu
from jax.experimental.pallas import tpu_sc as plsc

N_NODES = 10000
N_FEATS = 128
N_EDGES = 320000
N_LAYERS = 3

NC = 2
NS = 16
NW = NC * NS
CHUNK = 80
EPT = N_EDGES // NW
PAD_ROWS = 240
N_PAD = N_NODES + PAD_ROWS
CPT = (EPT + PAD_ROWS) // CHUNK
RPS = N_PAD // NS
DEGW = 128


def _vector_mesh():
    return plsc.VectorSubcoreMesh(core_axis_name="c", subcore_axis_name="s")


def _sc_degree(dst_t, zdeg, ones):
    @functools.partial(
        pl.kernel,
        out_type=jax.ShapeDtypeStruct((NC, N_PAD, DEGW), jnp.float32),
        mesh=_vector_mesh(),
        scratch_types=[
            pltpu.VMEM((CPT, CHUNK), jnp.int32),
            pltpu.VMEM((CHUNK, DEGW), jnp.float32),
            pltpu.VMEM_SHARED((N_PAD, DEGW), jnp.float32),
        ],
    )
    def k(dst_hbm, z_hbm, o_hbm, out_hbm, dstv, onesv, acc):
        cid = lax.axis_index("c")
        sid = lax.axis_index("s")
        wid = sid * NC + cid
        base = sid * RPS
        pltpu.sync_copy(z_hbm.at[pl.ds(base, RPS)], acc.at[pl.ds(base, RPS)])
        pltpu.sync_copy(o_hbm, onesv)
        pltpu.sync_copy(dst_hbm.at[wid], dstv)
        plsc.subcore_barrier()

        @pl.loop(0, CPT)
        def _(c):
            pltpu.sync_copy(onesv, acc.at[dstv.at[c]], add=True)

        plsc.subcore_barrier()
        pltpu.sync_copy(acc.at[pl.ds(base, RPS)],
                        out_hbm.at[cid, pl.ds(base, RPS)])

    return k(dst_t, zdeg, ones)


def _sc_scatter_rows(y, src_t, dst_t, zrows):
    @functools.partial(
        pl.kernel,
        out_type=jax.ShapeDtypeStruct((NC, N_PAD, N_FEATS), jnp.float32),
        mesh=_vector_mesh(),
        scratch_types=[
            pltpu.VMEM((CPT * CHUNK,), jnp.int32),
            pltpu.VMEM((CPT, CHUNK), jnp.int32),
            pltpu.VMEM((CHUNK, N_FEATS), jnp.float32),
            pltpu.VMEM((CHUNK, N_FEATS), jnp.float32),
            pltpu.VMEM_SHARED((N_PAD, N_FEATS), jnp.float32),
            pltpu.SemaphoreType.DMA,
            pltpu.SemaphoreType.DMA,
        ],
    )
    def k(y_hbm, src_hbm, dst_hbm, z_hbm, out_hbm,
          srcv, dstv, rows0, rows1, acc, sem0, sem1):
        cid = lax.axis_index("c")
        sid = lax.axis_index("s")
        wid = sid * NC + cid
        base = sid * RPS
        pltpu.sync_copy(z_hbm.at[pl.ds(base, RPS)], acc.at[pl.ds(base, RPS)])
        pltpu.sync_copy(src_hbm.at[wid], srcv)
        pltpu.sync_copy(dst_hbm.at[wid], dstv)
        plsc.subcore_barrier()

        pltpu.async_copy(y_hbm.at[srcv.at[pl.ds(0, CHUNK)]], rows0, sem0)
        pltpu.async_copy(y_hbm.at[srcv.at[pl.ds(CHUNK, CHUNK)]], rows1, sem1)

        @pl.loop(0, CPT, step=2)
        def _(c):
            pltpu.make_async_copy(
                y_hbm.at[srcv.at[pl.ds(c * CHUNK, CHUNK)]], rows0, sem0).wait()
            pltpu.sync_copy(rows0, acc.at[dstv.at[c]], add=True)

            @pl.when(c + 2 < CPT)
            def _():
                pltpu.async_copy(
                    y_hbm.at[srcv.at[pl.ds((c + 2) * CHUNK, CHUNK)]], rows0, sem0)

            pltpu.make_async_copy(
                y_hbm.at[srcv.at[pl.ds((c + 1) * CHUNK, CHUNK)]], rows1, sem1).wait()
            pltpu.sync_copy(rows1, acc.at[dstv.at[c + 1]], add=True)

            @pl.when(c + 3 < CPT)
            def _():
                pltpu.async_copy(
                    y_hbm.at[srcv.at[pl.ds((c + 3) * CHUNK, CHUNK)]], rows1, sem1)

        plsc.subcore_barrier()
        pltpu.sync_copy(acc.at[pl.ds(base, RPS)],
                        out_hbm.at[cid, pl.ds(base, RPS)])

    return k(y, src_t, dst_t, zrows)


_BR = 256
_GRID = (N_PAD // _BR,)


def _row_spec(w):
    return pl.BlockSpec((_BR, w), lambda i: (i, 0))


def _f32_out():
    return jax.ShapeDtypeStruct((N_PAD, N_FEATS), jnp.float32)


def _tc_prep(dega, degb, x0):
    def body(da, db, x, dis_o, y_o):
        deg = da[:, 0:1] + db[:, 0:1]
        dis = jnp.where(deg > 0.0, lax.rsqrt(jnp.maximum(deg, 1.0)), 0.0)
        disb = jnp.broadcast_to(dis, (_BR, N_FEATS))
        dis_o[...] = disb
        y_o[...] = x[...] * disb

    return pl.pallas_call(
        body,
        grid=_GRID,
        in_specs=[_row_spec(DEGW), _row_spec(DEGW), _row_spec(N_FEATS)],
        out_specs=[_row_spec(N_FEATS), _row_spec(N_FEATS)],
        out_shape=[_f32_out(), _f32_out()],
    )(dega, degb, x0)


def _tc_step(dis, sa, sb, accp):
    def body(d_ref, sa_ref, sb_ref, ap_ref, y_o, acc_o):
        d = d_ref[...]
        ds_ = d * (sa_ref[...] + sb_ref[...])
        y_o[...] = d * ds_
        acc_o[...] = ap_ref[...] + ds_

    return pl.pallas_call(
        body,
        grid=_GRID,
        in_specs=[_row_spec(N_FEATS)] * 4,
        out_specs=[_row_spec(N_FEATS), _row_spec(N_FEATS)],
        out_shape=[_f32_out(), _f32_out()],
    )(dis, sa, sb, accp)


def _tc_last(dis, sa, sb, accp, x0):
    def body(d_ref, sa_ref, sb_ref, ap_ref, x_ref, o_ref):
        ds_ = d_ref[...] * (sa_ref[...] + sb_ref[...])
        o_ref[...] = 0.25 * (x_ref[...] + ap_ref[...] + ds_)

    return pl.pallas_call(
        body,
        grid=_GRID,
        in_specs=[_row_spec(N_FEATS)] * 5,
        out_specs=_row_spec(N_FEATS),
        out_shape=_f32_out(),
    )(dis, sa, sb, accp, x0)


def kernel(edge_index_mp, emb_weight):
    pad = jnp.arange(N_NODES, N_NODES + PAD_ROWS, dtype=jnp.int32)
    padt = jnp.broadcast_to(pad, (NW, PAD_ROWS))
    src_t = jnp.concatenate(
        [edge_index_mp[0].reshape(NW, EPT), padt], axis=1
    )
    dst_t = jnp.concatenate(
        [edge_index_mp[1].reshape(NW, EPT), padt], axis=1
    ).reshape(NW, CPT, CHUNK)
    x0 = jnp.pad(emb_weight, ((0, PAD_ROWS), (0, 0)))
    zrows = jnp.zeros((N_PAD, N_FEATS), jnp.float32)
    zdeg = jnp.zeros((N_PAD, DEGW), jnp.float32)
    ones = jnp.ones((CHUNK, DEGW), jnp.float32)

    degp = _sc_degree(dst_t, zdeg, ones)
    dis, y = _tc_prep(degp[0], degp[1], x0)
    acc = zrows
    out = None
    for layer in range(N_LAYERS):
        s = _sc_scatter_rows(y, src_t, dst_t, zrows)
        if layer < N_LAYERS - 1:
            y, acc = _tc_step(dis, s[0], s[1], acc)
        else:
            out = _tc_last(dis, s[0], s[1], acc, x0)
    return out[:N_NODES]


# speedup vs baseline: 16.2367x; 1.1344x over previous
import dataclasses
import functools

import jax
import jax.numpy as jnp
from jax import lax
from jax.experimental import pallas as pl
from jax.experimental.pallas import tpu as pltpu
from jax.experimental.pallas import tpu_sc as plsc

N_NODES = 10000
N_FEATS = 128
N_EDGES = 320000
N_LAYERS = 3

NC = 2
NS = 16
NW = NC * NS
CHUNK = 80
EPT = N_EDGES // NW
PAD_ROWS = 240
N_PAD = N_NODES + PAD_ROWS
CPT = (EPT + PAD_ROWS) // CHUNK
RPS = N_PAD // NS


def _vector_mesh():
    return plsc.VectorSubcoreMesh(core_axis_name="c", subcore_axis_name="s")


def _sc_degree(dst_flat):
    cp = pltpu.CompilerParams()
    if "needs_layout_passes" in pltpu.CompilerParams.__dataclass_fields__:
        cp = dataclasses.replace(cp, needs_layout_passes=False)

    @functools.partial(
        pl.kernel,
        out_type=jax.ShapeDtypeStruct((NW, N_PAD), jnp.float32),
        mesh=_vector_mesh(),
        scratch_types=[
            pltpu.VMEM((CPT * CHUNK,), jnp.int32),
            pltpu.VMEM((N_PAD,), jnp.float32),
        ],
        compiler_params=cp,
    )
    def k(dst_hbm, out_hbm, dstf, hist):
        cid = lax.axis_index("c")
        sid = lax.axis_index("s")
        wid = sid * NC + cid
        zeros16 = jnp.zeros((16,), jnp.float32)
        ones16 = jnp.ones((16,), jnp.float32)

        @pl.loop(0, N_PAD, step=16)
        def _(i):
            hist[pl.ds(i, 16)] = zeros16

        pltpu.sync_copy(dst_hbm.at[wid], dstf)

        @pl.loop(0, CPT * CHUNK, step=16)
        def _(i):
            plsc.addupdate_scatter(hist, [dstf[pl.ds(i, 16)]], ones16)

        pltpu.sync_copy(hist, out_hbm.at[wid])

    return k(dst_flat)


def _sc_scatter_rows(y, src_t, dst_t, zrows):
    @functools.partial(
        pl.kernel,
        out_type=jax.ShapeDtypeStruct((NC, N_PAD, N_FEATS), jnp.float32),
        mesh=_vector_mesh(),
        scratch_types=[
            pltpu.VMEM((CPT * CHUNK,), jnp.int32),
            pltpu.VMEM((CPT, CHUNK), jnp.int32),
            pltpu.VMEM((CHUNK, N_FEATS), jnp.float32),
            pltpu.VMEM((CHUNK, N_FEATS), jnp.float32),
            pltpu.VMEM_SHARED((N_PAD, N_FEATS), jnp.float32),
            pltpu.SemaphoreType.DMA,
            pltpu.SemaphoreType.DMA,
        ],
    )
    def k(y_hbm, src_hbm, dst_hbm, z_hbm, out_hbm,
          srcv, dstv, rows0, rows1, acc, sem0, sem1):
        cid = lax.axis_index("c")
        sid = lax.axis_index("s")
        wid = sid * NC + cid
        base = sid * RPS
        pltpu.sync_copy(z_hbm.at[pl.ds(base, RPS)], acc.at[pl.ds(base, RPS)])
        pltpu.sync_copy(src_hbm.at[wid], srcv)
        pltpu.sync_copy(dst_hbm.at[wid], dstv)
        plsc.subcore_barrier()

        pltpu.async_copy(y_hbm.at[srcv.at[pl.ds(0, CHUNK)]], rows0, sem0)
        pltpu.async_copy(y_hbm.at[srcv.at[pl.ds(CHUNK, CHUNK)]], rows1, sem1)

        @pl.loop(0, CPT, step=2)
        def _(c):
            pltpu.make_async_copy(
                y_hbm.at[srcv.at[pl.ds(c * CHUNK, CHUNK)]], rows0, sem0).wait()
            pltpu.sync_copy(rows0, acc.at[dstv.at[c]], add=True)

            @pl.when(c + 2 < CPT)
            def _():
                pltpu.async_copy(
                    y_hbm.at[srcv.at[pl.ds((c + 2) * CHUNK, CHUNK)]], rows0, sem0)

            pltpu.make_async_copy(
                y_hbm.at[srcv.at[pl.ds((c + 1) * CHUNK, CHUNK)]], rows1, sem1).wait()
            pltpu.sync_copy(rows1, acc.at[dstv.at[c + 1]], add=True)

            @pl.when(c + 3 < CPT)
            def _():
                pltpu.async_copy(
                    y_hbm.at[srcv.at[pl.ds((c + 3) * CHUNK, CHUNK)]], rows1, sem1)

        plsc.subcore_barrier()
        pltpu.sync_copy(acc.at[pl.ds(base, RPS)],
                        out_hbm.at[cid, pl.ds(base, RPS)])

    return k(y, src_t, dst_t, zrows)


_BR = 256
_GRID = (N_PAD // _BR,)


def _row_spec(w):
    return pl.BlockSpec((_BR, w), lambda i: (i, 0))


def _f32_out():
    return jax.ShapeDtypeStruct((N_PAD, N_FEATS), jnp.float32)


def _tc_prep(degt, x0):
    def body(d_ref, x_ref, dis_o, y_o):
        deg = lax.dot_general(
            d_ref[...], jnp.ones((NW, 1), jnp.float32),
            dimension_numbers=(((0,), (0,)), ((), ())),
            preferred_element_type=jnp.float32,
        )
        dis = jnp.where(deg > 0.0, lax.rsqrt(jnp.maximum(deg, 1.0)), 0.0)
        disb = jnp.broadcast_to(dis, (_BR, N_FEATS))
        dis_o[...] = disb
        y_o[...] = x_ref[...] * disb

    return pl.pallas_call(
        body,
        grid=_GRID,
        in_specs=[pl.BlockSpec((NW, _BR), lambda i: (0, i)),
                  _row_spec(N_FEATS)],
        out_specs=[_row_spec(N_FEATS), _row_spec(N_FEATS)],
        out_shape=[_f32_out(), _f32_out()],
    )(degt, x0)


def _tc_step(dis, sa, sb, accp):
    def body(d_ref, sa_ref, sb_ref, ap_ref, y_o, acc_o):
        d = d_ref[...]
        ds_ = d * (sa_ref[...] + sb_ref[...])
        y_o[...] = d * ds_
        acc_o[...] = ap_ref[...] + ds_

    return pl.pallas_call(
        body,
        grid=_GRID,
        in_specs=[_row_spec(N_FEATS)] * 4,
        out_specs=[_row_spec(N_FEATS), _row_spec(N_FEATS)],
        out_shape=[_f32_out(), _f32_out()],
    )(dis, sa, sb, accp)


def _tc_last(dis, sa, sb, accp, x0):
    def body(d_ref, sa_ref, sb_ref, ap_ref, x_ref, o_ref):
        ds_ = d_ref[...] * (sa_ref[...] + sb_ref[...])
        o_ref[...] = 0.25 * (x_ref[...] + ap_ref[...] + ds_)

    return pl.pallas_call(
        body,
        grid=_GRID,
        in_specs=[_row_spec(N_FEATS)] * 5,
        out_specs=_row_spec(N_FEATS),
        out_shape=_f32_out(),
    )(dis, sa, sb, accp, x0)


def kernel(edge_index_mp, emb_weight):
    pad = jnp.arange(N_NODES, N_NODES + PAD_ROWS, dtype=jnp.int32)
    padt = jnp.broadcast_to(pad, (NW, PAD_ROWS))
    src_t = jnp.concatenate(
        [edge_index_mp[0].reshape(NW, EPT), padt], axis=1
    )
    dst_t = jnp.concatenate(
        [edge_index_mp[1].reshape(NW, EPT), padt], axis=1
    ).reshape(NW, CPT, CHUNK)
    dst_flat = jnp.concatenate(
        [edge_index_mp[1].reshape(NW, EPT), padt], axis=1
    )
    x0 = jnp.pad(emb_weight, ((0, PAD_ROWS), (0, 0)))
    zrows = jnp.zeros((N_PAD, N_FEATS), jnp.float32)

    degt = _sc_degree(dst_flat)
    dis, y = _tc_prep(degt, x0)
    acc = zrows
    out = None
    for layer in range(N_LAYERS):
        s = _sc_scatter_rows(y, src_t, dst_t, zrows)
        if layer < N_LAYERS - 1:
            y, acc = _tc_step(dis, s[0], s[1], acc)
        else:
            out = _tc_last(dis, s[0], s[1], acc, x0)
    return out[:N_NODES]


# speedup vs baseline: 16.6218x; 1.0237x over previous
import dataclasses
import functools

import jax
import jax.numpy as jnp
from jax import lax
from jax.experimental import pallas as pl
from jax.experimental.pallas import tpu as pltpu
from jax.experimental.pallas import tpu_sc as plsc

N_NODES = 10000
N_FEATS = 128
N_EDGES = 320000
N_LAYERS = 3

NC = 2
NS = 16
NW = NC * NS
CHUNK = 80
EPT = N_EDGES // NW
PAD_ROWS = 240
N_PAD = N_NODES + PAD_ROWS
CPT = (EPT + PAD_ROWS) // CHUNK
NBUF = 4
NSLOT = 16
RPS = N_PAD // NS


def _vector_mesh():
    return plsc.VectorSubcoreMesh(core_axis_name="c", subcore_axis_name="s")


def _sc_degree(dst_flat):
    cp = pltpu.CompilerParams()
    if "needs_layout_passes" in pltpu.CompilerParams.__dataclass_fields__:
        cp = dataclasses.replace(cp, needs_layout_passes=False)

    @functools.partial(
        pl.kernel,
        out_type=jax.ShapeDtypeStruct((NW, N_PAD), jnp.float32),
        mesh=_vector_mesh(),
        scratch_types=[
            pltpu.VMEM((CPT * CHUNK,), jnp.int32),
            pltpu.VMEM((N_PAD,), jnp.float32),
        ],
        compiler_params=cp,
    )
    def k(dst_hbm, out_hbm, dstf, hist):
        cid = lax.axis_index("c")
        sid = lax.axis_index("s")
        wid = sid * NC + cid
        zeros16 = jnp.zeros((16,), jnp.float32)
        ones16 = jnp.ones((16,), jnp.float32)

        @pl.loop(0, N_PAD, step=16)
        def _(i):
            hist[pl.ds(i, 16)] = zeros16

        pltpu.sync_copy(dst_hbm.at[wid], dstf)

        @pl.loop(0, CPT * CHUNK, step=16)
        def _(i):
            plsc.addupdate_scatter(hist, [dstf[pl.ds(i, 16)]], ones16)

        pltpu.sync_copy(hist, out_hbm.at[wid])

    return k(dst_flat)


def _sc_scatter_rows(y, src_flat, dst_t, zrows):
    @functools.partial(
        pl.kernel,
        out_type=jax.ShapeDtypeStruct((NC, N_PAD, N_FEATS), jnp.float32),
        mesh=_vector_mesh(),
        scratch_types=[
            pltpu.VMEM((NSLOT, CHUNK), jnp.int32),
            pltpu.VMEM((NSLOT, CHUNK), jnp.int32),
            pltpu.VMEM((CHUNK, N_FEATS), jnp.float32),
            pltpu.VMEM((CHUNK, N_FEATS), jnp.float32),
            pltpu.VMEM((CHUNK, N_FEATS), jnp.float32),
            pltpu.VMEM((CHUNK, N_FEATS), jnp.float32),
            pltpu.VMEM_SHARED((N_PAD, N_FEATS), jnp.float32),
            pltpu.SemaphoreType.DMA((NBUF,)),
            pltpu.SemaphoreType.DMA((NBUF,)),
            pltpu.SemaphoreType.DMA((NSLOT,)),
        ],
    )
    def k(y_hbm, src_hbm, dst_hbm, z_hbm, out_hbm,
          srcv, dstv, r0, r1, r2, r3, acc, semg, sems, semi):
        rows = [r0, r1, r2, r3]
        cid = lax.axis_index("c")
        sid = lax.axis_index("s")
        wid = sid * NC + cid
        base = sid * RPS
        pltpu.sync_copy(z_hbm.at[pl.ds(base, RPS)], acc.at[pl.ds(base, RPS)])

        def idx_load(chunk, slot):
            pltpu.async_copy(src_hbm.at[wid, chunk], srcv.at[slot],
                             semi.at[slot])
            pltpu.async_copy(dst_hbm.at[wid, chunk], dstv.at[slot],
                             semi.at[slot])

        def idx_wait(chunk, slot):
            pltpu.make_async_copy(src_hbm.at[wid, chunk], srcv.at[slot],
                                  semi.at[slot]).wait()
            pltpu.make_async_copy(dst_hbm.at[wid, chunk], dstv.at[slot],
                                  semi.at[slot]).wait()

        def gather_issue(slot, buf):
            pltpu.async_copy(y_hbm.at[srcv.at[slot]], rows[buf],
                             semg.at[buf])

        def gather_wait(slot, buf):
            pltpu.make_async_copy(y_hbm.at[srcv.at[slot]], rows[buf],
                                  semg.at[buf]).wait()

        def scatter_issue(slot, buf):
            pltpu.async_copy(rows[buf], acc.at[dstv.at[slot]], sems.at[buf],
                             add=True)

        def scatter_wait(slot, buf):
            pltpu.make_async_copy(rows[buf], acc.at[dstv.at[slot]],
                                  sems.at[buf]).wait()

        for kk in range(NSLOT):
            idx_load(kk, kk)
        for kk in range(2):
            idx_wait(kk, kk)
            gather_issue(kk, kk)

        plsc.subcore_barrier()

        for c in range(NSLOT):
            gather_wait(c % NSLOT, c % NBUF)
            scatter_issue(c % NSLOT, c % NBUF)
            if c >= 2:
                scatter_wait((c - 2) % NSLOT, (c - 2) % NBUF)
            idx_wait(c + 2, (c + 2) % NSLOT)
            gather_issue((c + 2) % NSLOT, (c + 2) % NBUF)
            if c >= 2:
                idx_load(c + 14, (c + 14) % NSLOT)

        @pl.loop(NSLOT, CPT, step=NSLOT)
        def _(cb):
            for off in range(NSLOT):
                c = cb + off
                gather_wait(off, off % NBUF)
                scatter_issue(off, off % NBUF)
                scatter_wait((off - 2) % NSLOT, (off - 2) % NBUF)

                @pl.when(c + 2 < CPT)
                def _():
                    idx_wait(c + 2, (off + 2) % NSLOT)
                    gather_issue((off + 2) % NSLOT, (off + 2) % NBUF)

                @pl.when(c + 14 < CPT)
                def _():
                    idx_load(c + 14, (off + 14) % NSLOT)

        scatter_wait((CPT - 2) % NSLOT, (CPT - 2) % NBUF)
        scatter_wait((CPT - 1) % NSLOT, (CPT - 1) % NBUF)
        plsc.subcore_barrier()
        pltpu.sync_copy(acc.at[pl.ds(base, RPS)],
                        out_hbm.at[cid, pl.ds(base, RPS)])

    return k(y, src_flat, dst_t, zrows)


_BR = 256
_GRID = (N_PAD // _BR,)


def _row_spec(w):
    return pl.BlockSpec((_BR, w), lambda i: (i, 0))


def _f32_out():
    return jax.ShapeDtypeStruct((N_PAD, N_FEATS), jnp.float32)


def _tc_prep(degt, x0):
    def body(d_ref, x_ref, dis_o, y_o):
        deg = lax.dot_general(
            d_ref[...], jnp.ones((NW, 1), jnp.float32),
            dimension_numbers=(((0,), (0,)), ((), ())),
            preferred_element_type=jnp.float32,
        )
        dis = jnp.where(deg > 0.0, lax.rsqrt(jnp.maximum(deg, 1.0)), 0.0)
        disb = jnp.broadcast_to(dis, (_BR, N_FEATS))
        dis_o[...] = disb
        y_o[...] = x_ref[...] * disb

    return pl.pallas_call(
        body,
        grid=_GRID,
        in_specs=[pl.BlockSpec((NW, _BR), lambda i: (0, i)),
                  _row_spec(N_FEATS)],
        out_specs=[_row_spec(N_FEATS), _row_spec(N_FEATS)],
        out_shape=[_f32_out(), _f32_out()],
    )(degt, x0)


def _tc_step(dis, sa, sb, accp):
    def body(d_ref, sa_ref, sb_ref, ap_ref, y_o, acc_o):
        d = d_ref[...]
        ds_ = d * (sa_ref[...] + sb_ref[...])
        y_o[...] = d * ds_
        acc_o[...] = ap_ref[...] + ds_

    return pl.pallas_call(
        body,
        grid=_GRID,
        in_specs=[_row_spec(N_FEATS)] * 4,
        out_specs=[_row_spec(N_FEATS), _row_spec(N_FEATS)],
        out_shape=[_f32_out(), _f32_out()],
    )(dis, sa, sb, accp)


def _tc_last(dis, sa, sb, accp, x0):
    def body(d_ref, sa_ref, sb_ref, ap_ref, x_ref, o_ref):
        ds_ = d_ref[...] * (sa_ref[...] + sb_ref[...])
        o_ref[...] = 0.25 * (x_ref[...] + ap_ref[...] + ds_)

    return pl.pallas_call(
        body,
        grid=_GRID,
        in_specs=[_row_spec(N_FEATS)] * 5,
        out_specs=_row_spec(N_FEATS),
        out_shape=_f32_out(),
    )(dis, sa, sb, accp, x0)


def kernel(edge_index_mp, emb_weight):
    pad = jnp.arange(N_NODES, N_NODES + PAD_ROWS, dtype=jnp.int32)
    padt = jnp.broadcast_to(pad, (NW, PAD_ROWS))
    src_t = jnp.concatenate(
        [edge_index_mp[0].reshape(NW, EPT), padt], axis=1
    ).reshape(NW, CPT, CHUNK)
    dst_t = jnp.concatenate(
        [edge_index_mp[1].reshape(NW, EPT), padt], axis=1
    ).reshape(NW, CPT, CHUNK)
    dst_flat = jnp.concatenate(
        [edge_index_mp[1].reshape(NW, EPT), padt], axis=1
    )
    x0 = jnp.pad(emb_weight, ((0, PAD_ROWS), (0, 0)))
    zrows = jnp.zeros((N_PAD, N_FEATS), jnp.float32)

    degt = _sc_degree(dst_flat)
    dis, y = _tc_prep(degt, x0)
    acc = zrows
    out = None
    for layer in range(N_LAYERS):
        s = _sc_scatter_rows(y, src_t, dst_t, zrows)
        if layer < N_LAYERS - 1:
            y, acc = _tc_step(dis, s[0], s[1], acc)
        else:
            out = _tc_last(dis, s[0], s[1], acc, x0)
    return out[:N_NODES]


# speedup vs baseline: 18.9418x; 1.1396x over previous
import dataclasses
import functools

import jax
import jax.numpy as jnp
from jax import lax
from jax.experimental import pallas as pl
from jax.experimental.pallas import tpu as pltpu
from jax.experimental.pallas import tpu_sc as plsc

N_NODES = 10000
N_FEATS = 128
N_EDGES = 320000
N_LAYERS = 3

NC = 2
NS = 16
NW = NC * NS
CHUNK = 80
EPT = N_EDGES // NW
PAD_ROWS = 240
N_PAD = N_NODES + PAD_ROWS
CPT = (EPT + PAD_ROWS) // CHUNK
NBUF = 4
NSLOT = 16
RPS = N_PAD // NS


def _vector_mesh():
    return plsc.VectorSubcoreMesh(core_axis_name="c", subcore_axis_name="s")


def _sc_degree(dst_flat):
    cp = pltpu.CompilerParams()
    if "needs_layout_passes" in pltpu.CompilerParams.__dataclass_fields__:
        cp = dataclasses.replace(cp, needs_layout_passes=False)

    @functools.partial(
        pl.kernel,
        out_type=jax.ShapeDtypeStruct((NW, N_PAD), jnp.float32),
        mesh=_vector_mesh(),
        scratch_types=[
            pltpu.VMEM((CPT * CHUNK,), jnp.int32),
            pltpu.VMEM((N_PAD,), jnp.float32),
        ],
        compiler_params=cp,
    )
    def k(dst_hbm, out_hbm, dstf, hist):
        cid = lax.axis_index("c")
        sid = lax.axis_index("s")
        wid = sid * NC + cid
        zeros16 = jnp.zeros((16,), jnp.float32)
        ones16 = jnp.ones((16,), jnp.float32)

        @pl.loop(0, N_PAD, step=16)
        def _(i):
            hist[pl.ds(i, 16)] = zeros16

        pltpu.sync_copy(dst_hbm.at[wid], dstf)

        @pl.loop(0, CPT * CHUNK, step=16)
        def _(i):
            plsc.addupdate_scatter(hist, [dstf[pl.ds(i, 16)]], ones16)

        pltpu.sync_copy(hist, out_hbm.at[wid])

    return k(dst_flat)


def _sc_scatter_rows(y, src_flat, dst_t, zrows):
    @functools.partial(
        pl.kernel,
        out_type=jax.ShapeDtypeStruct((NC, N_PAD, N_FEATS), jnp.float32),
        mesh=_vector_mesh(),
        scratch_types=[
            pltpu.VMEM((NSLOT, CHUNK), jnp.int32),
            pltpu.VMEM((NSLOT, CHUNK), jnp.int32),
            pltpu.VMEM((CHUNK, N_FEATS), jnp.float32),
            pltpu.VMEM((CHUNK, N_FEATS), jnp.float32),
            pltpu.VMEM((CHUNK, N_FEATS), jnp.float32),
            pltpu.VMEM((CHUNK, N_FEATS), jnp.float32),
            pltpu.VMEM_SHARED((N_PAD, N_FEATS), jnp.float32),
            pltpu.SemaphoreType.DMA((NBUF,)),
            pltpu.SemaphoreType.DMA((NBUF,)),
            pltpu.SemaphoreType.DMA((NSLOT,)),
        ],
    )
    def k(y_hbm, src_hbm, dst_hbm, z_hbm, out_hbm,
          srcv, dstv, r0, r1, r2, r3, acc, semg, sems, semi):
        rows = [r0, r1, r2, r3]
        cid = lax.axis_index("c")
        sid = lax.axis_index("s")
        wid = sid * NC + cid
        base = sid * RPS
        pltpu.sync_copy(z_hbm.at[pl.ds(base, RPS)], acc.at[pl.ds(base, RPS)])

        def idx_load(chunk, slot):
            pltpu.async_copy(src_hbm.at[wid, chunk], srcv.at[slot],
                             semi.at[slot])
            pltpu.async_copy(dst_hbm.at[wid, chunk], dstv.at[slot],
                             semi.at[slot])

        def idx_wait(chunk, slot):
            pltpu.make_async_copy(src_hbm.at[wid, chunk], srcv.at[slot],
                                  semi.at[slot]).wait()
            pltpu.make_async_copy(dst_hbm.at[wid, chunk], dstv.at[slot],
                                  semi.at[slot]).wait()

        def gather_issue(slot, buf):
            pltpu.async_copy(y_hbm.at[srcv.at[slot]], rows[buf],
                             semg.at[buf])

        def gather_wait(slot, buf):
            pltpu.make_async_copy(y_hbm.at[srcv.at[slot]], rows[buf],
                                  semg.at[buf]).wait()

        def scatter_issue(slot, buf):
            pltpu.async_copy(rows[buf], acc.at[dstv.at[slot]], sems.at[buf],
                             add=True)

        def scatter_wait(slot, buf):
            pltpu.make_async_copy(rows[buf], acc.at[dstv.at[slot]],
                                  sems.at[buf]).wait()

        for kk in range(NSLOT):
            idx_load(kk, kk)
        for kk in range(2):
            idx_wait(kk, kk)
            gather_issue(kk, kk)

        plsc.subcore_barrier()

        for c in range(NSLOT):
            gather_wait(c % NSLOT, c % NBUF)
            scatter_issue(c % NSLOT, c % NBUF)
            if c >= 2:
                scatter_wait((c - 2) % NSLOT, (c - 2) % NBUF)
            idx_wait(c + 2, (c + 2) % NSLOT)
            gather_issue((c + 2) % NSLOT, (c + 2) % NBUF)
            if c >= 2:
                idx_load(c + 14, (c + 14) % NSLOT)

        @pl.loop(NSLOT, CPT, step=NSLOT)
        def _(cb):
            for off in range(NSLOT):
                c = cb + off
                gather_wait(off, off % NBUF)
                scatter_issue(off, off % NBUF)
                scatter_wait((off - 2) % NSLOT, (off - 2) % NBUF)

                @pl.when(c + 2 < CPT)
                def _():
                    idx_wait(c + 2, (off + 2) % NSLOT)
                    gather_issue((off + 2) % NSLOT, (off + 2) % NBUF)

                @pl.when(c + 14 < CPT)
                def _():
                    idx_load(c + 14, (off + 14) % NSLOT)

        scatter_wait((CPT - 2) % NSLOT, (CPT - 2) % NBUF)
        scatter_wait((CPT - 1) % NSLOT, (CPT - 1) % NBUF)
        plsc.subcore_barrier()
        pltpu.sync_copy(acc.at[pl.ds(base, RPS)],
                        out_hbm.at[cid, pl.ds(base, RPS)])

    return k(y, src_flat, dst_t, zrows)


_BR = 2048
_GRID = (N_PAD // _BR,)


def _row_spec(w):
    return pl.BlockSpec((_BR, w), lambda i: (i, 0))


def _f32_out():
    return jax.ShapeDtypeStruct((N_PAD, N_FEATS), jnp.float32)


def _tc_prep(degt, x0):
    def body(d_ref, x_ref, dis_o, y_o):
        deg = lax.dot_general(
            d_ref[...], jnp.ones((NW, 1), jnp.float32),
            dimension_numbers=(((0,), (0,)), ((), ())),
            preferred_element_type=jnp.float32,
        )
        dis = jnp.where(deg > 0.0, lax.rsqrt(jnp.maximum(deg, 1.0)), 0.0)
        disb = jnp.broadcast_to(dis, (_BR, N_FEATS))
        dis_o[...] = disb
        y_o[...] = x_ref[...] * disb

    return pl.pallas_call(
        body,
        grid=_GRID,
        in_specs=[pl.BlockSpec((NW, _BR), lambda i: (0, i)),
                  _row_spec(N_FEATS)],
        out_specs=[_row_spec(N_FEATS), _row_spec(N_FEATS)],
        out_shape=[_f32_out(), _f32_out()],
    )(degt, x0)


def _tc_step(dis, sa, sb, accp):
    def body(d_ref, sa_ref, sb_ref, ap_ref, y_o, acc_o):
        d = d_ref[...]
        ds_ = d * (sa_ref[...] + sb_ref[...])
        y_o[...] = d * ds_
        acc_o[...] = ap_ref[...] + ds_

    return pl.pallas_call(
        body,
        grid=_GRID,
        in_specs=[_row_spec(N_FEATS)] * 4,
        out_specs=[_row_spec(N_FEATS), _row_spec(N_FEATS)],
        out_shape=[_f32_out(), _f32_out()],
    )(dis, sa, sb, accp)


def _tc_last(dis, sa, sb, accp, x0):
    def body(d_ref, sa_ref, sb_ref, ap_ref, x_ref, o_ref):
        ds_ = d_ref[...] * (sa_ref[...] + sb_ref[...])
        o_ref[...] = 0.25 * (x_ref[...] + ap_ref[...] + ds_)

    return pl.pallas_call(
        body,
        grid=_GRID,
        in_specs=[_row_spec(N_FEATS)] * 5,
        out_specs=_row_spec(N_FEATS),
        out_shape=_f32_out(),
    )(dis, sa, sb, accp, x0)


def kernel(edge_index_mp, emb_weight):
    pad = jnp.arange(N_NODES, N_NODES + PAD_ROWS, dtype=jnp.int32)
    padt = jnp.broadcast_to(pad, (NW, PAD_ROWS))
    src_t = jnp.concatenate(
        [edge_index_mp[0].reshape(NW, EPT), padt], axis=1
    ).reshape(NW, CPT, CHUNK)
    dst_t = jnp.concatenate(
        [edge_index_mp[1].reshape(NW, EPT), padt], axis=1
    ).reshape(NW, CPT, CHUNK)
    dst_flat = jnp.concatenate(
        [edge_index_mp[1].reshape(NW, EPT), padt], axis=1
    )
    x0 = jnp.pad(emb_weight, ((0, PAD_ROWS), (0, 0)))
    zrows = jnp.zeros((N_PAD, N_FEATS), jnp.float32)

    degt = _sc_degree(dst_flat)
    dis, y = _tc_prep(degt, x0)
    acc = zrows
    out = None
    for layer in range(N_LAYERS):
        s = _sc_scatter_rows(y, src_t, dst_t, zrows)
        if layer < N_LAYERS - 1:
            y, acc = _tc_step(dis, s[0], s[1], acc)
        else:
            out = _tc_last(dis, s[0], s[1], acc, x0)
    return out[:N_NODES]


# speedup vs baseline: 19.0551x; 1.0060x over previous
import dataclasses
import functools

import jax
import jax.numpy as jnp
from jax import lax
from jax.experimental import pallas as pl
from jax.experimental.pallas import tpu as pltpu
from jax.experimental.pallas import tpu_sc as plsc

N_NODES = 10000
N_FEATS = 128
N_EDGES = 320000
N_LAYERS = 3

NC = 2
NS = 16
NW = NC * NS
CHUNK = 80
EPT = N_EDGES // NW
PAD_ROWS = 240
N_PAD = N_NODES + PAD_ROWS
CPT = (EPT + PAD_ROWS) // CHUNK
NBUF = 4
NSLOT = 16
RPS = N_PAD // NS


def _vector_mesh():
    return plsc.VectorSubcoreMesh(core_axis_name="c", subcore_axis_name="s")


def _sc_degree(dst_flat):
    cp = pltpu.CompilerParams()
    if "needs_layout_passes" in pltpu.CompilerParams.__dataclass_fields__:
        cp = dataclasses.replace(cp, needs_layout_passes=False)

    @functools.partial(
        pl.kernel,
        out_type=jax.ShapeDtypeStruct((NW, N_PAD), jnp.float32),
        mesh=_vector_mesh(),
        scratch_types=[
            pltpu.VMEM((CPT * CHUNK,), jnp.int32),
            pltpu.VMEM((N_PAD,), jnp.float32),
        ],
        compiler_params=cp,
    )
    def k(dst_hbm, out_hbm, dstf, hist):
        cid = lax.axis_index("c")
        sid = lax.axis_index("s")
        wid = sid * NC + cid
        zeros16 = jnp.zeros((16,), jnp.float32)
        ones16 = jnp.ones((16,), jnp.float32)

        @pl.loop(0, N_PAD, step=16)
        def _(i):
            hist[pl.ds(i, 16)] = zeros16

        pltpu.sync_copy(dst_hbm.at[wid], dstf)

        @pl.loop(0, CPT * CHUNK, step=16)
        def _(i):
            plsc.addupdate_scatter(hist, [dstf[pl.ds(i, 16)]], ones16)

        pltpu.sync_copy(hist, out_hbm.at[wid])

    return k(dst_flat)


def _sc_scatter_rows(y, src_flat, dst_t, zrows):
    @functools.partial(
        pl.kernel,
        out_type=jax.ShapeDtypeStruct((NC, N_PAD, N_FEATS), jnp.float32),
        mesh=_vector_mesh(),
        scratch_types=[
            pltpu.VMEM((NSLOT, CHUNK), jnp.int32),
            pltpu.VMEM((NSLOT, CHUNK), jnp.int32),
            pltpu.VMEM((CHUNK, N_FEATS), jnp.float32),
            pltpu.VMEM((CHUNK, N_FEATS), jnp.float32),
            pltpu.VMEM((CHUNK, N_FEATS), jnp.float32),
            pltpu.VMEM((CHUNK, N_FEATS), jnp.float32),
            pltpu.VMEM_SHARED((N_PAD, N_FEATS), jnp.float32),
            pltpu.SemaphoreType.DMA((NBUF,)),
            pltpu.SemaphoreType.DMA((NBUF,)),
            pltpu.SemaphoreType.DMA((NSLOT,)),
        ],
    )
    def k(y_hbm, src_hbm, dst_hbm, z_hbm, out_hbm,
          srcv, dstv, r0, r1, r2, r3, acc, semg, sems, semi):
        rows = [r0, r1, r2, r3]
        cid = lax.axis_index("c")
        sid = lax.axis_index("s")
        wid = sid * NC + cid
        base = sid * RPS
        pltpu.sync_copy(z_hbm.at[pl.ds(base, RPS)], acc.at[pl.ds(base, RPS)])

        def idx_load(chunk, slot):
            pltpu.async_copy(src_hbm.at[wid, chunk], srcv.at[slot],
                             semi.at[slot])
            pltpu.async_copy(dst_hbm.at[wid, chunk], dstv.at[slot],
                             semi.at[slot])

        def idx_wait(chunk, slot):
            pltpu.make_async_copy(src_hbm.at[wid, chunk], srcv.at[slot],
                                  semi.at[slot]).wait()
            pltpu.make_async_copy(dst_hbm.at[wid, chunk], dstv.at[slot],
                                  semi.at[slot]).wait()

        def gather_issue(slot, buf):
            pltpu.async_copy(y_hbm.at[srcv.at[slot]], rows[buf],
                             semg.at[buf])

        def gather_wait(slot, buf):
            pltpu.make_async_copy(y_hbm.at[srcv.at[slot]], rows[buf],
                                  semg.at[buf]).wait()

        def scatter_issue(slot, buf):
            pltpu.async_copy(rows[buf], acc.at[dstv.at[slot]], sems.at[buf],
                             add=True)

        def scatter_wait(slot, buf):
            pltpu.make_async_copy(rows[buf], acc.at[dstv.at[slot]],
                                  sems.at[buf]).wait()

        for kk in range(NSLOT):
            idx_load(kk, kk)
        for kk in range(2):
            idx_wait(kk, kk)
            gather_issue(kk, kk)

        plsc.subcore_barrier()

        for c in range(NSLOT):
            gather_wait(c % NSLOT, c % NBUF)
            scatter_issue(c % NSLOT, c % NBUF)
            if c >= 2:
                scatter_wait((c - 2) % NSLOT, (c - 2) % NBUF)
            idx_wait(c + 2, (c + 2) % NSLOT)
            gather_issue((c + 2) % NSLOT, (c + 2) % NBUF)
            if c >= 2:
                idx_load(c + 14, (c + 14) % NSLOT)

        @pl.loop(NSLOT, CPT, step=NSLOT)
        def _(cb):
            for off in range(NSLOT):
                c = cb + off
                gather_wait(off, off % NBUF)
                scatter_issue(off, off % NBUF)
                scatter_wait((off - 2) % NSLOT, (off - 2) % NBUF)

                @pl.when(c + 2 < CPT)
                def _():
                    idx_wait(c + 2, (off + 2) % NSLOT)
                    gather_issue((off + 2) % NSLOT, (off + 2) % NBUF)

                @pl.when(c + 14 < CPT)
                def _():
                    idx_load(c + 14, (off + 14) % NSLOT)

        scatter_wait((CPT - 2) % NSLOT, (CPT - 2) % NBUF)
        scatter_wait((CPT - 1) % NSLOT, (CPT - 1) % NBUF)
        plsc.subcore_barrier()
        pltpu.sync_copy(acc.at[pl.ds(base, RPS)],
                        out_hbm.at[cid, pl.ds(base, RPS)])

    return k(y, src_flat, dst_t, zrows)


_BR = 5120
_GRID = (N_PAD // _BR,)


def _row_spec(w):
    return pl.BlockSpec((_BR, w), lambda i: (i, 0))


def _f32_out():
    return jax.ShapeDtypeStruct((N_PAD, N_FEATS), jnp.float32)


def _tc_prep(degt, x0):
    def body(d_ref, x_ref, dis_o, y_o):
        deg = lax.dot_general(
            d_ref[...], jnp.ones((NW, 1), jnp.float32),
            dimension_numbers=(((0,), (0,)), ((), ())),
            preferred_element_type=jnp.float32,
        )
        dis = jnp.where(deg > 0.0, lax.rsqrt(jnp.maximum(deg, 1.0)), 0.0)
        disb = jnp.broadcast_to(dis, (_BR, N_FEATS))
        dis_o[...] = disb
        y_o[...] = x_ref[...] * disb

    return pl.pallas_call(
        body,
        grid=_GRID,
        in_specs=[pl.BlockSpec((NW, _BR), lambda i: (0, i)),
                  _row_spec(N_FEATS)],
        out_specs=[_row_spec(N_FEATS), _row_spec(N_FEATS)],
        out_shape=[_f32_out(), _f32_out()],
    )(degt, x0)


def _tc_step(dis, sa, sb, accp):
    def body(d_ref, sa_ref, sb_ref, ap_ref, y_o, acc_o):
        d = d_ref[...]
        ds_ = d * (sa_ref[...] + sb_ref[...])
        y_o[...] = d * ds_
        acc_o[...] = ap_ref[...] + ds_

    return pl.pallas_call(
        body,
        grid=_GRID,
        in_specs=[_row_spec(N_FEATS)] * 4,
        out_specs=[_row_spec(N_FEATS), _row_spec(N_FEATS)],
        out_shape=[_f32_out(), _f32_out()],
    )(dis, sa, sb, accp)


def _tc_last(dis, sa, sb, accp, x0):
    def body(d_ref, sa_ref, sb_ref, ap_ref, x_ref, o_ref):
        ds_ = d_ref[...] * (sa_ref[...] + sb_ref[...])
        o_ref[...] = 0.25 * (x_ref[...] + ap_ref[...] + ds_)

    return pl.pallas_call(
        body,
        grid=_GRID,
        in_specs=[_row_spec(N_FEATS)] * 5,
        out_specs=_row_spec(N_FEATS),
        out_shape=_f32_out(),
    )(dis, sa, sb, accp, x0)


def kernel(edge_index_mp, emb_weight):
    pad = jnp.arange(N_NODES, N_NODES + PAD_ROWS, dtype=jnp.int32)
    padt = jnp.broadcast_to(pad, (NW, PAD_ROWS))
    src_t = jnp.concatenate(
        [edge_index_mp[0].reshape(NW, EPT), padt], axis=1
    ).reshape(NW, CPT, CHUNK)
    dst_t = jnp.concatenate(
        [edge_index_mp[1].reshape(NW, EPT), padt], axis=1
    ).reshape(NW, CPT, CHUNK)
    dst_flat = jnp.concatenate(
        [edge_index_mp[1].reshape(NW, EPT), padt], axis=1
    )
    x0 = jnp.pad(emb_weight, ((0, PAD_ROWS), (0, 0)))
    zrows = jnp.zeros((N_PAD, N_FEATS), jnp.float32)

    degt = _sc_degree(dst_flat)
    dis, y = _tc_prep(degt, x0)
    acc = zrows
    out = None
    for layer in range(N_LAYERS):
        s = _sc_scatter_rows(y, src_t, dst_t, zrows)
        if layer < N_LAYERS - 1:
            y, acc = _tc_step(dis, s[0], s[1], acc)
        else:
            out = _tc_last(dis, s[0], s[1], acc, x0)
    return out[:N_NODES]
